# baseline (speedup 1.0000x reference)
import jax
import jax.numpy as jnp
from jax import lax
from jax.experimental import pallas as pl
from jax.experimental.pallas import tpu as pltpu
from jax.experimental.pallas import tpu_sc as plsc

N_NODES = 10000
N_EDGES = 160000
N_COMM = 64
D_IN = 256
D_EMB = 64
N_CLS = 16

NC, NS = 2, 16
NP = 10240
ROWS_PER_TILE = NP // NS
CH = 128
EC0 = 58
EC1 = 22
ECT = EC0 + EC1
NPW = NP // (NC * NS)
PCH = 64
PK = NPW // PCH
DW = 128
EB = 4096
EHG = 40

_mesh = plsc.VectorSubcoreMesh(core_axis_name="c", subcore_axis_name="s")
_SC_PARAMS = pltpu.CompilerParams(needs_layout_passes=False)


def _f32(*shape):
    return jax.ShapeDtypeStruct(shape, jnp.float32)


def _sc_perm(permh, zb, zp, permv, gbuf, sem):
    c = lax.axis_index("c")
    s = lax.axis_index("s")
    wid = s * NC + c
    pltpu.sync_copy(permh.at[wid], permv)
    for k in range(PK):
        pltpu.async_copy(zb.at[permv.at[k]], gbuf, sem).wait()
        pltpu.sync_copy(gbuf, zp.at[pl.ds(wid * NPW + PCH * k, PCH)])


def _sc_prop(rowh0, colh0, rowh1, colh1, u, zeros_h, sp,
             rowv, colv, gbuf0, gbuf1, acc, semg0, semg1, sems0, sems1):
    c = lax.axis_index("c")
    s = lax.axis_index("s")
    for k in range(ROWS_PER_TILE // CH):
        pltpu.sync_copy(zeros_h, acc.at[pl.ds(s * ROWS_PER_TILE + CH * k, CH)])

    def edge_loop(rowh, colh, n):
        pltpu.sync_copy(rowh.at[s], rowv.at[pl.ds(0, n)])
        pltpu.sync_copy(colh.at[s], colv.at[pl.ds(0, n)])
        pltpu.async_copy(u.at[rowv.at[0]], gbuf0, semg0)
        pltpu.async_copy(u.at[rowv.at[1]], gbuf1, semg1)

        def step(jj, carry):
            j = 2 * jj
            nj = jnp.minimum(j + 2, n - 2)
            pltpu.make_async_copy(u.at[rowv.at[j]], gbuf0, semg0).wait()
            pltpu.async_copy(gbuf0, acc.at[colv.at[j]], sems0, add=True)
            pltpu.make_async_copy(u.at[rowv.at[j + 1]], gbuf1, semg1).wait()
            pltpu.async_copy(gbuf1, acc.at[colv.at[j + 1]], sems1, add=True)
            pltpu.make_async_copy(gbuf0, acc.at[colv.at[j]], sems0).wait()
            pltpu.async_copy(u.at[rowv.at[nj]], gbuf0, semg0)
            pltpu.make_async_copy(gbuf1, acc.at[colv.at[j + 1]], sems1).wait()
            pltpu.async_copy(u.at[rowv.at[nj + 1]], gbuf1, semg1)
            return carry

        lax.fori_loop(0, n // 2, step, 0)
        pltpu.make_async_copy(u.at[rowv.at[n - 2]], gbuf0, semg0).wait()
        pltpu.make_async_copy(u.at[rowv.at[n - 1]], gbuf1, semg1).wait()

    plsc.subcore_barrier()

    @pl.when(c == 0)
    def _():
        edge_loop(rowh0, colh0, EC0)

    @pl.when(c == 1)
    def _():
        edge_loop(rowh1, colh1, EC1)

    plsc.subcore_barrier()
    sl = pl.ds(s * ROWS_PER_TILE, ROWS_PER_TILE)
    pltpu.sync_copy(acc.at[sl], sp.at[c].at[sl])


def _sc_pool(emb2, lblh, zeros_h, gsum, lblv, embv, accg, sem):
    c = lax.axis_index("c")
    s = lax.axis_index("s")
    wid = s * NC + c
    pltpu.sync_copy(zeros_h.at[pl.ds(0, 8)], accg.at[pl.ds(8 * s, 8)])
    pltpu.sync_copy(lblh.at[wid], lblv)
    plsc.subcore_barrier()
    for k in range(PK):
        pltpu.sync_copy(emb2.at[pl.ds(wid * NPW + PCH * k, PCH)], embv)
        pltpu.sync_copy(embv, accg.at[lblv.at[k]], add=True)
    plsc.subcore_barrier()
    pltpu.sync_copy(accg.at[pl.ds(8 * s, 8)], gsum.at[c].at[pl.ds(8 * s, 8)])


_perm_call = pl.kernel(
    _sc_perm,
    out_type=_f32(NP, DW),
    mesh=_mesh,
    compiler_params=_SC_PARAMS,
    scratch_types=[
        pltpu.VMEM((PK, PCH), jnp.int32),
        pltpu.VMEM((PCH, DW), jnp.float32),
        pltpu.SemaphoreType.DMA,
    ],
)

_prop_call = pl.kernel(
    _sc_prop,
    out_type=_f32(NC, NP, DW),
    mesh=_mesh,
    compiler_params=_SC_PARAMS,
    scratch_types=[
        pltpu.VMEM((EC0, CH), jnp.int32),
        pltpu.VMEM((EC0, CH), jnp.int32),
        pltpu.VMEM((CH, DW), jnp.float32),
        pltpu.VMEM((CH, DW), jnp.float32),
        pltpu.VMEM_SHARED((NP, DW), jnp.float32),
        pltpu.SemaphoreType.DMA,
        pltpu.SemaphoreType.DMA,
        pltpu.SemaphoreType.DMA,
        pltpu.SemaphoreType.DMA,
    ],
)

_pool_call = pl.kernel(
    _sc_pool,
    out_type=_f32(NC, 128, DW),
    mesh=_mesh,
    compiler_params=_SC_PARAMS,
    scratch_types=[
        pltpu.VMEM((PK, PCH), jnp.int32),
        pltpu.VMEM((PCH, DW), jnp.float32),
        pltpu.VMEM_SHARED((128, DW), jnp.float32),
        pltpu.SemaphoreType.DMA,
    ],
)


_BLK = 1024
_GRID = NP // _BLK


def _lane_iota(shape):
    return lax.broadcasted_iota(jnp.int32, shape, 1)


def _tc_mm(xr, wr, outr):
    outr[...] = jnp.dot(xr[...], wr[...], preferred_element_type=jnp.float32)


def _tc_deghist(colr, outr):
    i = pl.program_id(0)

    @pl.when(i == 0)
    def _():
        outr[...] = jnp.zeros((128, 128), jnp.float32)

    c = colr[0, 0]
    il = _lane_iota((EB, 128))
    ohlo = ((c & 127)[:, None] == il).astype(jnp.bfloat16)
    ohhiT = ((c >> 7)[None, :] == lax.broadcasted_iota(
        jnp.int32, (128, EB), 0)).astype(jnp.bfloat16)
    outr[...] += jnp.dot(ohhiT, ohlo, preferred_element_type=jnp.float32)


def _tc_u0(dgr, zbr, zpr, u0r, d1r, d2r):
    ri = lax.broadcasted_iota(jnp.int32, (_BLK, 8), 0)
    sel = ((ri >> 7) == lax.broadcasted_iota(jnp.int32, (_BLK, 8), 1))
    t1 = jnp.dot(sel.astype(jnp.float32), dgr[...],
                 preferred_element_type=jnp.float32)
    m = _lane_iota((_BLK, 128)) == (
        lax.broadcasted_iota(jnp.int32, (_BLK, 128), 0) & 127)
    deg = jnp.sum(jnp.where(m, t1, 0.0), axis=1)
    d1b = jnp.broadcast_to(lax.rsqrt(deg + 1.0)[:, None], (_BLK, DW))
    d2b = jnp.broadcast_to(lax.rsqrt(deg + 2.0)[:, None], (_BLK, DW))
    d1r[...] = d1b
    d2r[...] = d2b
    lane = _lane_iota((_BLK, DW)) < D_EMB
    h0 = jnp.where(lane, zbr[...], zpr[...])
    u0r[...] = h0 * d1b


def _tc_u1(d1r, s1r, u0r, u1r):
    d1 = d1r[...]
    s1 = s1r[...]
    u1r[...] = (s1[0] + s1[1] + u0r[...]) * (d1 * d1)


def _tc_u2(d1r, d2r, s2r, u1r, b1r, w2r, u2r):
    s2 = s2r[...]
    h2 = (s2[0] + s2[1] + u1r[...]) * d1r[...] + b1r[...]
    m = jnp.dot(h2, w2r[...], preferred_element_type=jnp.float32)
    u2r[...] = m * d2r[...]


def _tc_h4(d2r, s3r, u2r, b2r, wcr, h4r, embr, logr):
    s3 = s3r[...]
    h4 = (s3[0] + s3[1] + 2.0 * u2r[...]) * d2r[...] + b2r[...]
    h4r[...] = h4
    ci = _lane_iota((_BLK, DW))
    embr[...] = jnp.where(ci < D_EMB, jnp.maximum(h4, 0.0),
                          jnp.where(ci == D_EMB, 1.0, 0.0))
    logr[...] = jnp.dot(h4, wcr[...], preferred_element_type=jnp.float32)


def _tc_readout(gsr, wrr, brr, h4r, posr, negr):
    gs = gsr[...]
    g128 = gs[0] + gs[1]
    ci = _lane_iota((128, DW))
    cnt = jnp.sum(jnp.where(ci == D_EMB, g128, 0.0), axis=1)
    gnorm = g128 / jnp.maximum(cnt, 1.0)[:, None]
    ri = lax.broadcasted_iota(jnp.int32, (128, DW), 0)
    gbar = jnp.sum(jnp.where(ri < N_COMM, gnorm, 0.0), axis=0) / N_COMM
    v = gbar[:D_EMB] * wrr[...][0]
    b = brr[0, 0]
    h4 = h4r[...]
    p = jnp.sum(jnp.maximum(h4[:, :D_EMB], 0.0) * v[None, :], axis=1) + b
    q = jnp.sum(h4[:, D_EMB:] * v[None, :], axis=1) + b
    posr[...] = jnp.broadcast_to(jax.nn.sigmoid(p)[:, None], (_BLK, 8))
    negr[...] = jnp.broadcast_to(jax.nn.sigmoid(q)[:, None], (_BLK, 8))


def _row_spec(width):
    return pl.BlockSpec((_BLK, width), lambda i: (i, 0))


def _s_spec():
    return pl.BlockSpec((NC, _BLK, DW), lambda i: (0, i, 0))


def _full_spec(shape):
    nd = len(shape)
    return pl.BlockSpec(shape, lambda i, _nd=nd: (0,) * _nd)


def kernel(x, edge_index, community_labels, W1, b1, W2, b2, Wc, Wr, br):
    row = edge_index[0].astype(jnp.int32)
    col = edge_index[1].astype(jnp.int32)

    e_pad = NS * ECT * CH - N_EDGES
    rowf = jnp.pad(row, (0, e_pad), constant_values=0)
    pad_cols = N_NODES + (jnp.arange(e_pad, dtype=jnp.int32)
                          % (NP - N_NODES))
    colf = jnp.concatenate([col, pad_cols])
    n0 = NS * EC0 * CH
    rowp0 = rowf[:n0].reshape(NS, EC0, CH)
    colp0 = colf[:n0].reshape(NS, EC0, CH)
    rowp1 = rowf[n0:].reshape(NS, EC1, CH)
    colp1 = colf[n0:].reshape(NS, EC1, CH)
    colh = jnp.pad(col, (0, EHG * EB - N_EDGES),
                   constant_values=N_NODES + 100).reshape(EHG, 1, EB)
    perm = jax.random.permutation(jax.random.key(1), N_NODES).astype(jnp.int32)
    permp = jnp.pad(perm, (0, NP - N_NODES)).reshape(NC * NS, PK, PCH)
    lblp = jnp.pad(community_labels.astype(jnp.int32), (0, NP - N_NODES),
                   constant_values=N_COMM).reshape(NC * NS, PK, PCH)
    zeros_h = jnp.zeros((CH, DW), jnp.float32)
    b1_2 = jnp.tile(b1, 2).reshape(1, DW)
    b2_2 = jnp.tile(b2, 2).reshape(1, DW)
    wr_2 = Wr.reshape(1, D_EMB)
    br_2 = br.reshape(1, 1)
    w1d = jnp.concatenate([W1, W1], axis=1)
    w2d = jnp.zeros((DW, DW), jnp.float32)
    w2d = w2d.at[:D_EMB, :D_EMB].set(W2).at[D_EMB:, D_EMB:].set(W2)
    wcf = jnp.zeros((DW, 128), jnp.float32).at[:D_EMB, :N_CLS].set(Wc)

    zb = pl.pallas_call(
        _tc_mm,
        grid=(_GRID,),
        in_specs=[_row_spec(D_IN), _full_spec((D_IN, DW))],
        out_specs=_row_spec(DW),
        out_shape=_f32(NP, DW),
    )(x, w1d)

    deg2d = pl.pallas_call(
        _tc_deghist,
        grid=(EHG,),
        in_specs=[pl.BlockSpec((1, 1, EB), lambda i: (i, 0, 0))],
        out_specs=_full_spec((128, 128)),
        out_shape=_f32(128, 128),
    )(colh)

    zp = _perm_call(permp, zb)

    u0, d1b, d2b = pl.pallas_call(
        _tc_u0,
        grid=(_GRID,),
        in_specs=[pl.BlockSpec((8, 128), lambda i: (i, 0)),
                  _row_spec(DW), _row_spec(DW)],
        out_specs=[_row_spec(DW), _row_spec(DW), _row_spec(DW)],
        out_shape=[_f32(NP, DW), _f32(NP, DW), _f32(NP, DW)],
    )(deg2d, zb, zp)

    s1 = _prop_call(rowp0, colp0, rowp1, colp1, u0, zeros_h)

    u1 = pl.pallas_call(
        _tc_u1,
        grid=(_GRID,),
        in_specs=[_row_spec(DW), _s_spec(), _row_spec(DW)],
        out_specs=_row_spec(DW),
        out_shape=_f32(NP, DW),
    )(d1b, s1, u0)

    s2 = _prop_call(rowp0, colp0, rowp1, colp1, u1, zeros_h)

    u2 = pl.pallas_call(
        _tc_u2,
        grid=(_GRID,),
        in_specs=[_row_spec(DW), _row_spec(DW), _s_spec(), _row_spec(DW),
                  _full_spec((1, DW)), _full_spec((DW, DW))],
        out_specs=_row_spec(DW),
        out_shape=_f32(NP, DW),
    )(d1b, d2b, s2, u1, b1_2, w2d)

    s3 = _prop_call(rowp0, colp0, rowp1, colp1, u2, zeros_h)

    h4, emb2, logits_p = pl.pallas_call(
        _tc_h4,
        grid=(_GRID,),
        in_specs=[_row_spec(DW), _s_spec(), _row_spec(DW),
                  _full_spec((1, DW)), _full_spec((DW, 128))],
        out_specs=[_row_spec(DW), _row_spec(DW), _row_spec(128)],
        out_shape=[_f32(NP, DW), _f32(NP, DW), _f32(NP, 128)],
    )(d2b, s3, u2, b2_2, wcf)

    gsum = _pool_call(emb2, lblp, zeros_h)

    posb, negb = pl.pallas_call(
        _tc_readout,
        grid=(_GRID,),
        in_specs=[_full_spec((NC, 128, DW)),
                  _full_spec((1, D_EMB)), _full_spec((1, 1)),
                  _row_spec(DW)],
        out_specs=[_row_spec(8), _row_spec(8)],
        out_shape=[_f32(NP, 8), _f32(NP, 8)],
    )(gsum, wr_2, br_2, h4)

    logits = logits_p[:N_NODES, :N_CLS]
    pos = posb[:N_NODES, :1]
    neg = negb[:N_NODES, :1]
    return logits, pos, neg


# speedup vs baseline: 1.1232x; 1.1232x over previous
import jax
import jax.numpy as jnp
from jax import lax
from jax.experimental import pallas as pl
from jax.experimental.pallas import tpu as pltpu
from jax.experimental.pallas import tpu_sc as plsc

N_NODES = 10000
N_EDGES = 160000
N_COMM = 64
D_IN = 256
D_EMB = 64
N_CLS = 16

NC, NS = 2, 16
NP = 10240
ROWS_PER_TILE = NP // NS
CH = 128
EC0 = 58
EC1 = 22
ECT = EC0 + EC1
NPW = NP // (NC * NS)
PCH = 64
PK = NPW // PCH
DW = 128
EB = 4096
EHG = 40

_mesh = plsc.VectorSubcoreMesh(core_axis_name="c", subcore_axis_name="s")
_SC_PARAMS = pltpu.CompilerParams(needs_layout_passes=False)


def _f32(*shape):
    return jax.ShapeDtypeStruct(shape, jnp.float32)


def _sc_perm(permh, zb, zp, permv, gbuf, sem):
    c = lax.axis_index("c")
    s = lax.axis_index("s")
    wid = s * NC + c
    pltpu.sync_copy(permh.at[wid], permv)
    for k in range(PK):
        pltpu.async_copy(zb.at[permv.at[k]], gbuf, sem).wait()
        pltpu.sync_copy(gbuf, zp.at[pl.ds(wid * NPW + PCH * k, PCH)])


def _sc_prop(rowh0, colh0, rowh1, colh1, u, zeros_h, sp,
             rowv, colv, gbuf0, gbuf1, acc, semg0, semg1, sems0, sems1):
    c = lax.axis_index("c")
    s = lax.axis_index("s")
    for k in range(ROWS_PER_TILE // CH):
        pltpu.sync_copy(zeros_h, acc.at[pl.ds(s * ROWS_PER_TILE + CH * k, CH)])

    def edge_loop(rowh, colh, n):
        pltpu.sync_copy(rowh.at[s], rowv.at[pl.ds(0, n)])
        pltpu.sync_copy(colh.at[s], colv.at[pl.ds(0, n)])
        pltpu.async_copy(u.at[rowv.at[0]], gbuf0, semg0)
        pltpu.async_copy(u.at[rowv.at[1]], gbuf1, semg1)

        def step(jj, carry):
            j = 2 * jj
            nj = jnp.minimum(j + 2, n - 2)
            pltpu.make_async_copy(u.at[rowv.at[j]], gbuf0, semg0).wait()
            pltpu.async_copy(gbuf0, acc.at[colv.at[j]], sems0, add=True)
            pltpu.make_async_copy(u.at[rowv.at[j + 1]], gbuf1, semg1).wait()
            pltpu.async_copy(gbuf1, acc.at[colv.at[j + 1]], sems1, add=True)
            pltpu.make_async_copy(gbuf0, acc.at[colv.at[j]], sems0).wait()
            pltpu.async_copy(u.at[rowv.at[nj]], gbuf0, semg0)
            pltpu.make_async_copy(gbuf1, acc.at[colv.at[j + 1]], sems1).wait()
            pltpu.async_copy(u.at[rowv.at[nj + 1]], gbuf1, semg1)
            return carry

        lax.fori_loop(0, n // 2, step, 0)
        pltpu.make_async_copy(u.at[rowv.at[n - 2]], gbuf0, semg0).wait()
        pltpu.make_async_copy(u.at[rowv.at[n - 1]], gbuf1, semg1).wait()

    plsc.subcore_barrier()

    @pl.when(c == 0)
    def _():
        edge_loop(rowh0, colh0, EC0)

    @pl.when(c == 1)
    def _():
        edge_loop(rowh1, colh1, EC1)

    plsc.subcore_barrier()
    sl = pl.ds(s * ROWS_PER_TILE, ROWS_PER_TILE)
    pltpu.sync_copy(acc.at[sl], sp.at[c].at[sl])


def _sc_pool(emb2, lblh, zeros_h, gsum, lblv, embv, accg, sem):
    c = lax.axis_index("c")
    s = lax.axis_index("s")
    wid = s * NC + c
    pltpu.sync_copy(zeros_h.at[pl.ds(0, 8)], accg.at[pl.ds(8 * s, 8)])
    pltpu.sync_copy(lblh.at[wid], lblv)
    plsc.subcore_barrier()
    for k in range(PK):
        pltpu.sync_copy(emb2.at[pl.ds(wid * NPW + PCH * k, PCH)], embv)
        pltpu.sync_copy(embv, accg.at[lblv.at[k]], add=True)
    plsc.subcore_barrier()
    pltpu.sync_copy(accg.at[pl.ds(8 * s, 8)], gsum.at[c].at[pl.ds(8 * s, 8)])


_perm_call = pl.kernel(
    _sc_perm,
    out_type=_f32(NP, DW),
    mesh=_mesh,
    compiler_params=_SC_PARAMS,
    scratch_types=[
        pltpu.VMEM((PK, PCH), jnp.int32),
        pltpu.VMEM((PCH, DW), jnp.float32),
        pltpu.SemaphoreType.DMA,
    ],
)

_prop_call = pl.kernel(
    _sc_prop,
    out_type=_f32(NC, NP, DW),
    mesh=_mesh,
    compiler_params=_SC_PARAMS,
    scratch_types=[
        pltpu.VMEM((EC0, CH), jnp.int32),
        pltpu.VMEM((EC0, CH), jnp.int32),
        pltpu.VMEM((CH, DW), jnp.float32),
        pltpu.VMEM((CH, DW), jnp.float32),
        pltpu.VMEM_SHARED((NP, DW), jnp.float32),
        pltpu.SemaphoreType.DMA,
        pltpu.SemaphoreType.DMA,
        pltpu.SemaphoreType.DMA,
        pltpu.SemaphoreType.DMA,
    ],
)

_pool_call = pl.kernel(
    _sc_pool,
    out_type=_f32(NC, 128, DW),
    mesh=_mesh,
    compiler_params=_SC_PARAMS,
    scratch_types=[
        pltpu.VMEM((PK, PCH), jnp.int32),
        pltpu.VMEM((PCH, DW), jnp.float32),
        pltpu.VMEM_SHARED((128, DW), jnp.float32),
        pltpu.SemaphoreType.DMA,
    ],
)


_BLK = 1024
_GRID = NP // _BLK


def _lane_iota(shape):
    return lax.broadcasted_iota(jnp.int32, shape, 1)


def _tc_mm(xr, wr, outr):
    outr[...] = jnp.dot(xr[...], wr[...], preferred_element_type=jnp.float32)


def _tc_deghist(colr, outr):
    i = pl.program_id(0)

    @pl.when(i == 0)
    def _():
        outr[...] = jnp.zeros((128, 128), jnp.float32)

    c = colr[0, 0]
    il = _lane_iota((EB, 128))
    ohlo = ((c & 127)[:, None] == il).astype(jnp.float32)
    ohhiT = ((c >> 7)[None, :] == lax.broadcasted_iota(
        jnp.int32, (128, EB), 0)).astype(jnp.float32)
    outr[...] += jnp.dot(ohhiT, ohlo, preferred_element_type=jnp.float32)


def _tc_degexpand(dgr, d1r, d2r):
    ri = lax.broadcasted_iota(jnp.int32, (_BLK, 8), 0)
    sel = ((ri >> 7) == lax.broadcasted_iota(jnp.int32, (_BLK, 8), 1))
    t1 = jnp.dot(sel.astype(jnp.float32), dgr[...],
                 preferred_element_type=jnp.float32)
    m = _lane_iota((_BLK, 128)) == (
        lax.broadcasted_iota(jnp.int32, (_BLK, 128), 0) & 127)
    deg = jnp.sum(jnp.where(m, t1, 0.0), axis=1)
    d1r[...] = jnp.broadcast_to(lax.rsqrt(deg + 1.0)[:, None], (_BLK, DW))
    d2r[...] = jnp.broadcast_to(lax.rsqrt(deg + 2.0)[:, None], (_BLK, DW))


def _tc_u0(d1r, zbr, zpr, u0r):
    lane = _lane_iota((_BLK, DW)) < D_EMB
    h0 = jnp.where(lane, zbr[...], zpr[...])
    u0r[...] = h0 * d1r[...]


def _tc_u1(d1r, s1r, u0r, u1r):
    d1 = d1r[...]
    s1 = s1r[...]
    u1r[...] = (s1[0] + s1[1] + u0r[...]) * (d1 * d1)


def _tc_u2(d1r, d2r, s2r, u1r, b1r, w2r, u2r):
    s2 = s2r[...]
    h2 = (s2[0] + s2[1] + u1r[...]) * d1r[...] + b1r[...]
    m = jnp.dot(h2, w2r[...], preferred_element_type=jnp.float32)
    u2r[...] = m * d2r[...]


def _tc_h4(d2r, s3r, u2r, b2r, wcr, h4r, embr, logr):
    s3 = s3r[...]
    h4 = (s3[0] + s3[1] + 2.0 * u2r[...]) * d2r[...] + b2r[...]
    h4r[...] = h4
    ci = _lane_iota((_BLK, DW))
    embr[...] = jnp.where(ci < D_EMB, jnp.maximum(h4, 0.0),
                          jnp.where(ci == D_EMB, 1.0, 0.0))
    logr[...] = jnp.dot(h4, wcr[...], preferred_element_type=jnp.float32)


def _tc_readout(gsr, wrr, brr, h4r, posr, negr):
    gs = gsr[...]
    g128 = gs[0] + gs[1]
    ci = _lane_iota((128, DW))
    cnt = jnp.sum(jnp.where(ci == D_EMB, g128, 0.0), axis=1)
    gnorm = g128 / jnp.maximum(cnt, 1.0)[:, None]
    ri = lax.broadcasted_iota(jnp.int32, (128, DW), 0)
    gbar = jnp.sum(jnp.where(ri < N_COMM, gnorm, 0.0), axis=0) / N_COMM
    v = gbar[:D_EMB] * wrr[...][0]
    b = brr[0, 0]
    h4 = h4r[...]
    p = jnp.sum(jnp.maximum(h4[:, :D_EMB], 0.0) * v[None, :], axis=1) + b
    q = jnp.sum(h4[:, D_EMB:] * v[None, :], axis=1) + b
    posr[...] = jnp.broadcast_to(jax.nn.sigmoid(p)[:, None], (_BLK, 8))
    negr[...] = jnp.broadcast_to(jax.nn.sigmoid(q)[:, None], (_BLK, 8))


def _row_spec(width):
    return pl.BlockSpec((_BLK, width), lambda i: (i, 0))


def _s_spec():
    return pl.BlockSpec((NC, _BLK, DW), lambda i: (0, i, 0))


def _full_spec(shape):
    nd = len(shape)
    return pl.BlockSpec(shape, lambda i, _nd=nd: (0,) * _nd)


def kernel(x, edge_index, community_labels, W1, b1, W2, b2, Wc, Wr, br):
    row = edge_index[0].astype(jnp.int32)
    col = edge_index[1].astype(jnp.int32)

    x_pad = jnp.pad(x, ((0, NP - N_NODES), (0, 0)))

    e_pad = NS * ECT * CH - N_EDGES
    rowf = jnp.pad(row, (0, e_pad), constant_values=0)
    pad_cols = N_NODES + (jnp.arange(e_pad, dtype=jnp.int32)
                          % (NP - N_NODES))
    colf = jnp.concatenate([col, pad_cols])
    n0 = NS * EC0 * CH
    rowp0 = rowf[:n0].reshape(NS, EC0, CH)
    colp0 = colf[:n0].reshape(NS, EC0, CH)
    rowp1 = rowf[n0:].reshape(NS, EC1, CH)
    colp1 = colf[n0:].reshape(NS, EC1, CH)
    colh = jnp.pad(col, (0, EHG * EB - N_EDGES),
                   constant_values=N_NODES + 100).reshape(EHG, 1, EB)
    perm = jax.random.permutation(jax.random.key(1), N_NODES).astype(jnp.int32)
    permp = jnp.pad(perm, (0, NP - N_NODES)).reshape(NC * NS, PK, PCH)
    lblp = jnp.pad(community_labels.astype(jnp.int32), (0, NP - N_NODES),
                   constant_values=N_COMM).reshape(NC * NS, PK, PCH)
    zeros_h = jnp.zeros((CH, DW), jnp.float32)
    b1_2 = jnp.tile(b1, 2).reshape(1, DW)
    b2_2 = jnp.tile(b2, 2).reshape(1, DW)
    wr_2 = Wr.reshape(1, D_EMB)
    br_2 = br.reshape(1, 1)
    w1d = jnp.concatenate([W1, W1], axis=1)
    w2d = jnp.zeros((DW, DW), jnp.float32)
    w2d = w2d.at[:D_EMB, :D_EMB].set(W2).at[D_EMB:, D_EMB:].set(W2)
    wcf = jnp.zeros((DW, 128), jnp.float32).at[:D_EMB, :N_CLS].set(Wc)

    zb = pl.pallas_call(
        _tc_mm,
        grid=(_GRID,),
        in_specs=[_row_spec(D_IN), _full_spec((D_IN, DW))],
        out_specs=_row_spec(DW),
        out_shape=_f32(NP, DW),
    )(x_pad, w1d)

    deg2d = pl.pallas_call(
        _tc_deghist,
        grid=(EHG,),
        in_specs=[pl.BlockSpec((1, 1, EB), lambda i: (i, 0, 0))],
        out_specs=_full_spec((128, 128)),
        out_shape=_f32(128, 128),
    )(colh)

    d1b, d2b = pl.pallas_call(
        _tc_degexpand,
        grid=(_GRID,),
        in_specs=[pl.BlockSpec((8, 128), lambda i: (i, 0))],
        out_specs=[_row_spec(DW), _row_spec(DW)],
        out_shape=[_f32(NP, DW), _f32(NP, DW)],
    )(deg2d)

    zp = _perm_call(permp, zb)

    u0 = pl.pallas_call(
        _tc_u0,
        grid=(_GRID,),
        in_specs=[_row_spec(DW), _row_spec(DW), _row_spec(DW)],
        out_specs=_row_spec(DW),
        out_shape=_f32(NP, DW),
    )(d1b, zb, zp)

    s1 = _prop_call(rowp0, colp0, rowp1, colp1, u0, zeros_h)

    u1 = pl.pallas_call(
        _tc_u1,
        grid=(_GRID,),
        in_specs=[_row_spec(DW), _s_spec(), _row_spec(DW)],
        out_specs=_row_spec(DW),
        out_shape=_f32(NP, DW),
    )(d1b, s1, u0)

    s2 = _prop_call(rowp0, colp0, rowp1, colp1, u1, zeros_h)

    u2 = pl.pallas_call(
        _tc_u2,
        grid=(_GRID,),
        in_specs=[_row_spec(DW), _row_spec(DW), _s_spec(), _row_spec(DW),
                  _full_spec((1, DW)), _full_spec((DW, DW))],
        out_specs=_row_spec(DW),
        out_shape=_f32(NP, DW),
    )(d1b, d2b, s2, u1, b1_2, w2d)

    s3 = _prop_call(rowp0, colp0, rowp1, colp1, u2, zeros_h)

    h4, emb2, logits_p = pl.pallas_call(
        _tc_h4,
        grid=(_GRID,),
        in_specs=[_row_spec(DW), _s_spec(), _row_spec(DW),
                  _full_spec((1, DW)), _full_spec((DW, 128))],
        out_specs=[_row_spec(DW), _row_spec(DW), _row_spec(128)],
        out_shape=[_f32(NP, DW), _f32(NP, DW), _f32(NP, 128)],
    )(d2b, s3, u2, b2_2, wcf)

    gsum = _pool_call(emb2, lblp, zeros_h)

    posb, negb = pl.pallas_call(
        _tc_readout,
        grid=(_GRID,),
        in_specs=[_full_spec((NC, 128, DW)),
                  _full_spec((1, D_EMB)), _full_spec((1, 1)),
                  _row_spec(DW)],
        out_specs=[_row_spec(8), _row_spec(8)],
        out_shape=[_f32(NP, 8), _f32(NP, 8)],
    )(gsum, wr_2, br_2, h4)

    logits = logits_p[:N_NODES, :N_CLS]
    pos = posb[:N_NODES, :1]
    neg = negb[:N_NODES, :1]
    return logits, pos, neg
